# baseline (speedup 1.0000x reference)
import functools

import jax
import jax.numpy as jnp
from jax import lax
from jax.experimental import pallas as pl
from jax.experimental.pallas import tpu as pltpu
from jax.experimental.pallas import tpu_sc as plsc

B, P, L = 1024, 20, 8
NHID = 128
IMG_FEA = 512
MVOC = 10000
REG = 0.001

NC, NS = 2, 16
NW = NC * NS
U = 2 * B * P
UPT = U // NW
G = 8
NGRP = UPT // G
NIMG = 3 * B
IQT = B // NW


@functools.lru_cache(maxsize=None)
def _make_sc_gather():
    mesh = plsc.VectorSubcoreMesh(core_axis_name="c", subcore_axis_name="s",
                                  num_cores=NC, num_subcores=NS)

    @functools.partial(
        pl.kernel,
        out_type=(
            jax.ShapeDtypeStruct((U, NHID), jnp.float32),
            jax.ShapeDtypeStruct((NIMG, IMG_FEA), jnp.float32),
        ),
        mesh=mesh,
        scratch_types=[
            pltpu.VMEM_SHARED((MVOC, NHID), jnp.float32),
            pltpu.VMEM((NGRP // 2, G * L), jnp.int32),
            pltpu.VMEM((NGRP // 2, G * L), jnp.float32),
            pltpu.VMEM((2, G * L, NHID), jnp.float32),
            pltpu.VMEM((2, G, NHID), jnp.float32),
            pltpu.VMEM((IQT // 2,), jnp.int32),
            pltpu.VMEM((IQT // 2, IMG_FEA), jnp.float32),
            pltpu.SemaphoreType.DMA,
            pltpu.SemaphoreType.DMA,
            pltpu.SemaphoreType.DMA,
            pltpu.SemaphoreType.DMA,
            pltpu.SemaphoreType.DMA,
        ],
    )
    def _sc_gather(table_hbm, ppath_hbm, npath_hbm, pmask_hbm, nmask_hbm,
                   imgt_hbm, qid_hbm, pid_hbm, nid_hbm,
                   res_hbm, irows_hbm,
                   table_sh, idx_all, mask_all, rows2, out2, iidx_v, irows_v,
                   gsem0, gsem1, osem0, osem1, isem):
        wid = lax.axis_index("s") * NC + lax.axis_index("c")
        tix = lax.rem(wid, NW // 2)
        sub = lax.axis_index("s")

        seg = 1000

        @pl.when(sub < MVOC // seg)
        def _stage_table():
            pltpu.sync_copy(table_hbm.at[pl.ds(sub * seg, seg)],
                            table_sh.at[pl.ds(sub * seg, seg)])

        plsc.subcore_barrier()

        qbase = wid * IQT
        IH = IQT // 2
        for j, id_hbm in enumerate((qid_hbm, pid_hbm, nid_hbm)):
            for h in range(2):
                pltpu.sync_copy(id_hbm.at[pl.ds(qbase + h * IH, IH)], iidx_v)
                pltpu.async_copy(imgt_hbm.at[iidx_v], irows_v, isem).wait()
                pltpu.sync_copy(
                    irows_v,
                    irows_hbm.at[pl.ds(j * B + qbase + h * IH, IH)])

        HGRP = NGRP // 2
        ubase = wid * UPT

        def compute_group(g, buf):
            for u2 in range(G // 2):
                mv = mask_all[g, pl.ds(u2 * 16, 16)]
                for k in range(2):
                    u = 2 * u2 + k
                    m = [mv[k * L + l] for l in range(L)]
                    av, bv, cv = [], [], []
                    for i in range(L // 2):
                        me, mo = m[2 * i], m[2 * i + 1]
                        b = mo * mo
                        av.append(jnp.full((16,), me, jnp.float32))
                        bv.append(jnp.full((16,), b, jnp.float32))
                        cv.append(jnp.full((16,), 1.0 - me + mo - b,
                                           jnp.float32))
                    for c in range(NHID // 16):
                        sl = pl.ds(c * 16, 16)
                        r_ = [rows2[buf, u * L + l, sl] for l in range(L)]
                        meta = [r_[2 * i] * av[i]
                                + (r_[2 * i + 1] * bv[i] + cv[i])
                                for i in range(L // 2)]
                        out2[buf, u, sl] = (meta[1] * (meta[0] + meta[2])
                                            + meta[2] * meta[3])

        def gather(g, buf, sem):
            src = table_sh if buf == 0 else table_hbm
            return pltpu.async_copy(src.at[idx_all.at[g]],
                                    rows2.at[buf], sem)

        def gather_wait(g, buf, sem):
            src = table_sh if buf == 0 else table_hbm
            pltpu.make_async_copy(src.at[idx_all.at[g]],
                                  rows2.at[buf], sem).wait()

        def out_drain(buf, sem):
            pltpu.make_async_copy(out2.at[buf], res_hbm.at[pl.ds(ubase, G)],
                                  sem).wait()

        def phase(ph, carry):
            pg = ph * HGRP

            @pl.when(wid < NW // 2)
            def _stage_pos():
                pltpu.sync_copy(ppath_hbm.at[tix, pl.ds(pg, HGRP)], idx_all)
                pltpu.sync_copy(pmask_hbm.at[tix, pl.ds(pg, HGRP)], mask_all)

            @pl.when(wid >= NW // 2)
            def _stage_neg():
                pltpu.sync_copy(npath_hbm.at[tix, pl.ds(pg, HGRP)], idx_all)
                pltpu.sync_copy(nmask_hbm.at[tix, pl.ds(pg, HGRP)], mask_all)

            pbase = ubase + pg * G
            gather(0, 0, gsem0)

            def pair(gp, carry2):
                a = 2 * gp
                gather(a + 1, 1, gsem1)

                @pl.when(gp > 0)
                def _drain_prev():
                    out_drain(0, osem0)
                    out_drain(1, osem1)

                gather_wait(a, 0, gsem0)
                compute_group(a, 0)
                pltpu.async_copy(out2.at[0],
                                 res_hbm.at[pl.ds(pbase + a * G, G)], osem0)

                @pl.when(a + 2 < HGRP)
                def _next_gather():
                    gather(a + 2, 0, gsem0)

                gather_wait(a + 1, 1, gsem1)
                compute_group(a + 1, 1)
                pltpu.async_copy(out2.at[1],
                                 res_hbm.at[pl.ds(pbase + (a + 1) * G, G)],
                                 osem1)
                return carry2

            lax.fori_loop(0, HGRP // 2, pair, 0)
            out_drain(0, osem0)
            out_drain(1, osem1)
            return carry

        lax.fori_loop(0, 2, phase, 0)

    return _sc_gather


BB = 128
NSTEP = B // BB
MROWS = MVOC // NSTEP


def _tc_body(irows_ref, w_ref, b_ref, res_ref, leaf_ref, hw_ref, hb_ref,
             mt_ref, out_ref, acc_ref):
    i = pl.program_id(0)

    @pl.when(i == 0)
    def _init():
        for k in range(5):
            acc_ref[k] = 0.0

    irows = irows_ref[...].reshape(3 * BB, IMG_FEA)
    proj = lax.dot_general(irows, w_ref[...], (((1,), (1,)), ((), ())),
                           preferred_element_type=jnp.float32)
    proj = proj + b_ref[...]
    proj = proj.reshape(3, BB, NHID)
    q, pI, nI = proj[0], proj[1], proj[2]

    res = res_ref[...]
    ss = jnp.sum(res * res, axis=-1, keepdims=True)
    pe = res / jnp.maximum(jnp.sqrt(ss), 1e-12)

    hw = hw_ref[...]
    hb = hb_ref[0, 0]
    leaf = leaf_ref[...]

    def pool(pe_s, leaf_s, user, item):
        uim = user * item
        uis = user - item
        fusion = uim[:, None, :] - uis[:, None, :] * pe_s
        w = jnp.sum(fusion * hw[None], axis=-1) + hb
        w = w * (1.0 / (1.0 + jnp.exp(-2.0 * leaf_s)))
        w = w - jnp.max(w, axis=-1, keepdims=True)
        e = jnp.exp(w)
        w = e / jnp.sum(e, axis=-1, keepdims=True)
        return jnp.sum(pe_s * w[..., None], axis=1)

    pPool = pool(pe[0], leaf[0], q, pI)
    nPool = pool(pe[1], leaf[1], q, nI)
    ps = jnp.sum(q * pI + (pI - q) * pPool, axis=1)
    ns = jnp.sum(q * nI + (nI - q) * nPool, axis=1)
    step_loss = jnp.sum(jnp.log(1.0 + jnp.exp(ns - ps)))

    mt = mt_ref[...]
    acc_ref[0] += step_loss
    acc_ref[1] += jnp.sum(mt * mt)
    acc_ref[2] += jnp.sum(q * q)
    acc_ref[3] += jnp.sum(pI * pI)
    acc_ref[4] += jnp.sum(nI * nI)
    total = acc_ref[0] + REG * (
        jnp.sqrt(acc_ref[1]) + jnp.sqrt(acc_ref[2])
        + jnp.sqrt(acc_ref[3]) + jnp.sqrt(acc_ref[4]))
    out_ref[...] = jnp.full((1, NHID), total, dtype=jnp.float32)


_tc_call = pl.pallas_call(
    _tc_body,
    grid=(NSTEP,),
    in_specs=[
        pl.BlockSpec((3, BB, IMG_FEA), lambda i: (0, i, 0)),
        pl.BlockSpec((NHID, IMG_FEA), lambda i: (0, 0)),
        pl.BlockSpec((1, NHID), lambda i: (0, 0)),
        pl.BlockSpec((2, BB, P, NHID), lambda i: (0, i, 0, 0)),
        pl.BlockSpec((2, BB, P), lambda i: (0, i, 0)),
        pl.BlockSpec((1, NHID), lambda i: (0, 0)),
        pl.BlockSpec((1, 1), lambda i: (0, 0)),
        pl.BlockSpec((1, MROWS, NHID), lambda i: (i, 0, 0)),
    ],
    out_specs=pl.BlockSpec((1, NHID), lambda i: (0, 0)),
    out_shape=jax.ShapeDtypeStruct((1, NHID), jnp.float32),
    scratch_shapes=[pltpu.SMEM((8,), jnp.float32)],
)


def kernel(qry_id, pos_id, neg_id, pos_path, pos_mask, pos_leafnodeMask,
           neg_path, neg_mask, neg_leafnodeMask, img_features, imageW_w,
           imageW_b, meta_table, h_att_w, h_att_b):
    res_all, img_rows = _make_sc_gather()(
        meta_table,
        pos_path.astype(jnp.int32).reshape(NW // 2, NGRP, G * L),
        neg_path.astype(jnp.int32).reshape(NW // 2, NGRP, G * L),
        pos_mask.reshape(NW // 2, NGRP, G * L),
        neg_mask.reshape(NW // 2, NGRP, G * L),
        img_features,
        qry_id.astype(jnp.int32).reshape(B),
        pos_id.astype(jnp.int32).reshape(B),
        neg_id.astype(jnp.int32).reshape(B))

    out = _tc_call(
        img_rows.reshape(3, B, IMG_FEA),
        imageW_w,
        imageW_b.reshape(1, NHID),
        res_all.reshape(2, B, P, NHID),
        jnp.stack([pos_leafnodeMask, neg_leafnodeMask]),
        h_att_w,
        h_att_b.reshape(1, 1),
        meta_table.reshape(NSTEP, MROWS, NHID),
    )
    return out[0, 0]


# speedup vs baseline: 1.1916x; 1.1916x over previous
import functools

import jax
import jax.numpy as jnp
from jax import lax
from jax.experimental import pallas as pl
from jax.experimental.pallas import tpu as pltpu
from jax.experimental.pallas import tpu_sc as plsc

B, P, L = 1024, 20, 8
NHID = 128
IMG_FEA = 512
MVOC = 10000
REG = 0.001

NC, NS = 2, 16
NW = NC * NS
U = 2 * B * P
UPT = U // NW
G = 8
NGRP = UPT // G
NIMG = 3 * B
IQT = B // NW


@functools.lru_cache(maxsize=None)
def _make_sc_gather():
    mesh = plsc.VectorSubcoreMesh(core_axis_name="c", subcore_axis_name="s",
                                  num_cores=NC, num_subcores=NS)

    @functools.partial(
        pl.kernel,
        out_type=(
            jax.ShapeDtypeStruct((U, NHID), jnp.float32),
            jax.ShapeDtypeStruct((NIMG, IMG_FEA), jnp.float32),
        ),
        mesh=mesh,
        scratch_types=[
            pltpu.VMEM_SHARED((MVOC, NHID), jnp.float32),
            pltpu.VMEM((NGRP // 2, G * L), jnp.int32),
            pltpu.VMEM((NGRP // 2, G * L), jnp.float32),
            pltpu.VMEM((2, G * L, NHID), jnp.float32),
            pltpu.VMEM((2, G, NHID), jnp.float32),
            pltpu.VMEM((IQT // 2,), jnp.int32),
            pltpu.VMEM((IQT // 2, IMG_FEA), jnp.float32),
            pltpu.SemaphoreType.DMA,
            pltpu.SemaphoreType.DMA,
            pltpu.SemaphoreType.DMA,
            pltpu.SemaphoreType.DMA,
            pltpu.SemaphoreType.DMA,
        ],
    )
    def _sc_gather(table_hbm, ppath_hbm, npath_hbm, pmask_hbm, nmask_hbm,
                   imgt_hbm, qid_hbm, pid_hbm, nid_hbm,
                   res_hbm, irows_hbm,
                   table_sh, idx_all, mask_all, rows2, out2, iidx_v, irows_v,
                   gsem0, gsem1, osem0, osem1, isem):
        wid = lax.axis_index("s") * NC + lax.axis_index("c")
        tix = lax.rem(wid, NW // 2)
        sub = lax.axis_index("s")

        seg = 1000

        @pl.when(sub < MVOC // seg)
        def _stage_table():
            pltpu.sync_copy(table_hbm.at[pl.ds(sub * seg, seg)],
                            table_sh.at[pl.ds(sub * seg, seg)])

        plsc.subcore_barrier()

        qbase = wid * IQT
        IH = IQT // 2
        for j, id_hbm in enumerate((qid_hbm, pid_hbm, nid_hbm)):
            for h in range(2):
                pltpu.sync_copy(id_hbm.at[pl.ds(qbase + h * IH, IH)], iidx_v)
                pltpu.async_copy(imgt_hbm.at[iidx_v], irows_v, isem).wait()
                pltpu.sync_copy(
                    irows_v,
                    irows_hbm.at[pl.ds(j * B + qbase + h * IH, IH)])

        HGRP = NGRP // 2
        ubase = wid * UPT

        def compute_group(g, buf):
            for u2 in range(G // 2):
                mv = mask_all[g, pl.ds(u2 * 16, 16)]
                for k in range(2):
                    u = 2 * u2 + k
                    m = [mv[k * L + l] for l in range(L)]
                    av, bv, cv = [], [], []
                    for i in range(L // 2):
                        me, mo = m[2 * i], m[2 * i + 1]
                        b = mo * mo
                        av.append(jnp.full((16,), me, jnp.float32))
                        bv.append(jnp.full((16,), b, jnp.float32))
                        cv.append(jnp.full((16,), 1.0 - me + mo - b,
                                           jnp.float32))
                    for c in range(NHID // 16):
                        sl = pl.ds(c * 16, 16)
                        r_ = [rows2[buf, u * L + l, sl] for l in range(L)]
                        meta = [r_[2 * i] * av[i]
                                + (r_[2 * i + 1] * bv[i] + cv[i])
                                for i in range(L // 2)]
                        out2[buf, u, sl] = (meta[1] * (meta[0] + meta[2])
                                            + meta[2] * meta[3])

        def gather(g, buf, sem):
            return pltpu.async_copy(table_sh.at[idx_all.at[g]],
                                    rows2.at[buf], sem)

        def gather_wait(g, buf, sem):
            pltpu.make_async_copy(table_sh.at[idx_all.at[g]],
                                  rows2.at[buf], sem).wait()

        def out_drain(buf, sem):
            pltpu.make_async_copy(out2.at[buf], res_hbm.at[pl.ds(ubase, G)],
                                  sem).wait()

        def phase(ph, carry):
            pg = ph * HGRP

            @pl.when(wid < NW // 2)
            def _stage_pos():
                pltpu.sync_copy(ppath_hbm.at[tix, pl.ds(pg, HGRP)], idx_all)
                pltpu.sync_copy(pmask_hbm.at[tix, pl.ds(pg, HGRP)], mask_all)

            @pl.when(wid >= NW // 2)
            def _stage_neg():
                pltpu.sync_copy(npath_hbm.at[tix, pl.ds(pg, HGRP)], idx_all)
                pltpu.sync_copy(nmask_hbm.at[tix, pl.ds(pg, HGRP)], mask_all)

            pbase = ubase + pg * G
            gather(0, 0, gsem0)

            def pair(gp, carry2):
                a = 2 * gp
                gather(a + 1, 1, gsem1)

                @pl.when(gp > 0)
                def _drain_prev():
                    out_drain(0, osem0)
                    out_drain(1, osem1)

                gather_wait(a, 0, gsem0)
                compute_group(a, 0)
                pltpu.async_copy(out2.at[0],
                                 res_hbm.at[pl.ds(pbase + a * G, G)], osem0)

                @pl.when(a + 2 < HGRP)
                def _next_gather():
                    gather(a + 2, 0, gsem0)

                gather_wait(a + 1, 1, gsem1)
                compute_group(a + 1, 1)
                pltpu.async_copy(out2.at[1],
                                 res_hbm.at[pl.ds(pbase + (a + 1) * G, G)],
                                 osem1)
                return carry2

            lax.fori_loop(0, HGRP // 2, pair, 0)
            out_drain(0, osem0)
            out_drain(1, osem1)
            return carry

        lax.fori_loop(0, 2, phase, 0)

    return _sc_gather


BB = 128
NSTEP = B // BB
MROWS = MVOC // NSTEP


def _tc_body(irows_ref, w_ref, b_ref, res_ref, leaf_ref, hw_ref, hb_ref,
             mt_ref, out_ref, acc_ref):
    i = pl.program_id(0)

    @pl.when(i == 0)
    def _init():
        for k in range(5):
            acc_ref[k] = 0.0

    irows = irows_ref[...].reshape(3 * BB, IMG_FEA)
    proj = lax.dot_general(irows, w_ref[...], (((1,), (1,)), ((), ())),
                           preferred_element_type=jnp.float32)
    proj = proj + b_ref[...]
    proj = proj.reshape(3, BB, NHID)
    q, pI, nI = proj[0], proj[1], proj[2]

    res = res_ref[...]
    ss = jnp.sum(res * res, axis=-1, keepdims=True)
    pe = res / jnp.maximum(jnp.sqrt(ss), 1e-12)

    hw = hw_ref[...]
    hb = hb_ref[0, 0]
    leaf = leaf_ref[...]

    def pool(pe_s, leaf_s, user, item):
        uim = user * item
        uis = user - item
        fusion = uim[:, None, :] - uis[:, None, :] * pe_s
        w = jnp.sum(fusion * hw[None], axis=-1) + hb
        w = w * (1.0 / (1.0 + jnp.exp(-2.0 * leaf_s)))
        w = w - jnp.max(w, axis=-1, keepdims=True)
        e = jnp.exp(w)
        w = e / jnp.sum(e, axis=-1, keepdims=True)
        return jnp.sum(pe_s * w[..., None], axis=1)

    pPool = pool(pe[0], leaf[0], q, pI)
    nPool = pool(pe[1], leaf[1], q, nI)
    ps = jnp.sum(q * pI + (pI - q) * pPool, axis=1)
    ns = jnp.sum(q * nI + (nI - q) * nPool, axis=1)
    step_loss = jnp.sum(jnp.log(1.0 + jnp.exp(ns - ps)))

    mt = mt_ref[...]
    acc_ref[0] += step_loss
    acc_ref[1] += jnp.sum(mt * mt)
    acc_ref[2] += jnp.sum(q * q)
    acc_ref[3] += jnp.sum(pI * pI)
    acc_ref[4] += jnp.sum(nI * nI)
    total = acc_ref[0] + REG * (
        jnp.sqrt(acc_ref[1]) + jnp.sqrt(acc_ref[2])
        + jnp.sqrt(acc_ref[3]) + jnp.sqrt(acc_ref[4]))
    out_ref[...] = jnp.full((1, NHID), total, dtype=jnp.float32)


_tc_call = pl.pallas_call(
    _tc_body,
    grid=(NSTEP,),
    in_specs=[
        pl.BlockSpec((3, BB, IMG_FEA), lambda i: (0, i, 0)),
        pl.BlockSpec((NHID, IMG_FEA), lambda i: (0, 0)),
        pl.BlockSpec((1, NHID), lambda i: (0, 0)),
        pl.BlockSpec((2, BB, P, NHID), lambda i: (0, i, 0, 0)),
        pl.BlockSpec((2, BB, P), lambda i: (0, i, 0)),
        pl.BlockSpec((1, NHID), lambda i: (0, 0)),
        pl.BlockSpec((1, 1), lambda i: (0, 0)),
        pl.BlockSpec((1, MROWS, NHID), lambda i: (i, 0, 0)),
    ],
    out_specs=pl.BlockSpec((1, NHID), lambda i: (0, 0)),
    out_shape=jax.ShapeDtypeStruct((1, NHID), jnp.float32),
    scratch_shapes=[pltpu.SMEM((8,), jnp.float32)],
)


def kernel(qry_id, pos_id, neg_id, pos_path, pos_mask, pos_leafnodeMask,
           neg_path, neg_mask, neg_leafnodeMask, img_features, imageW_w,
           imageW_b, meta_table, h_att_w, h_att_b):
    res_all, img_rows = _make_sc_gather()(
        meta_table,
        pos_path.astype(jnp.int32).reshape(NW // 2, NGRP, G * L),
        neg_path.astype(jnp.int32).reshape(NW // 2, NGRP, G * L),
        pos_mask.reshape(NW // 2, NGRP, G * L),
        neg_mask.reshape(NW // 2, NGRP, G * L),
        img_features,
        qry_id.astype(jnp.int32).reshape(B),
        pos_id.astype(jnp.int32).reshape(B),
        neg_id.astype(jnp.int32).reshape(B))

    out = _tc_call(
        img_rows.reshape(3, B, IMG_FEA),
        imageW_w,
        imageW_b.reshape(1, NHID),
        res_all.reshape(2, B, P, NHID),
        jnp.stack([pos_leafnodeMask, neg_leafnodeMask]),
        h_att_w,
        h_att_b.reshape(1, 1),
        meta_table.reshape(NSTEP, MROWS, NHID),
    )
    return out[0, 0]
